# baseline (speedup 1.0000x reference)
import jax
import jax.numpy as jnp
from jax import lax
from jax.experimental import pallas as pl
from jax.experimental.pallas import tpu as pltpu
from jax.experimental.pallas import tpu_sc as plsc

DIM = 64
SCALE = 8.0

NC = 2
NS = 16
NW = NC * NS

CHUNK = 128
LANES = 16
NBUF = 5


def _body(x_hbm, table_hbm, out_hbm, idx_v, in_bufs, out_bufs,
          sems_in, sems_out):
    wid = lax.axis_index("s") * NC + lax.axis_index("c")
    n_chunks = x_hbm.shape[1]
    row_base = wid * (n_chunks * CHUNK)

    pltpu.sync_copy(x_hbm.at[wid], idx_v)

    def gather(j, b):
        return pltpu.make_async_copy(
            table_hbm.at[idx_v.at[j]], in_bufs[b], sems_in[b])

    def write(j, b):
        return pltpu.make_async_copy(
            out_bufs[b], out_hbm.at[pl.ds(row_base + j * CHUNK, CHUNK)],
            sems_out[b])

    for b in range(NBUF):
        gather(b, b).start()

    def wave(w, _):
        for b in range(NBUF):
            j = w * NBUF + b
            gather(j, b).wait()

            @pl.when(w > 0)
            def _():
                write(j - NBUF, b).wait()

            def scale_row(i, _):
                for c in range(DIM // LANES):
                    sl = pl.ds(c * LANES, LANES)
                    out_bufs[b][i, sl] = in_bufs[b][i, sl] * SCALE
                return 0
            lax.fori_loop(0, CHUNK, scale_row, 0, unroll=8)

            write(j, b).start()

            @pl.when(j + NBUF < n_chunks)
            def _():
                gather(j + NBUF, b).start()
        return 0

    lax.fori_loop(0, n_chunks // NBUF, wave, 0)

    for b in range(NBUF):
        write(n_chunks - NBUF + b, b).wait()


def kernel(x, table):
    b0, b1 = x.shape
    total = b0 * b1
    n_chunks = total // (NW * CHUNK)
    xf = x.reshape(NW, n_chunks, CHUNK).astype(jnp.int32)

    mesh = plsc.VectorSubcoreMesh(core_axis_name="c", subcore_axis_name="s")
    run = pl.kernel(
        _body,
        out_type=jax.ShapeDtypeStruct((total, DIM), jnp.float32),
        mesh=mesh,
        scratch_types=[
            pltpu.VMEM((n_chunks, CHUNK), jnp.int32),
            [pltpu.VMEM((CHUNK, DIM), jnp.float32) for _ in range(NBUF)],
            [pltpu.VMEM((CHUNK, DIM), jnp.float32) for _ in range(NBUF)],
            [pltpu.SemaphoreType.DMA for _ in range(NBUF)],
            [pltpu.SemaphoreType.DMA for _ in range(NBUF)],
        ],
        compiler_params=pltpu.CompilerParams(use_tc_tiling_on_sc=False),
    )
    out = run(xf, table)
    return out.reshape(b0, b1, DIM)


# speedup vs baseline: 1.2681x; 1.2681x over previous
import jax
import jax.numpy as jnp
from jax import lax
from jax.experimental import pallas as pl
from jax.experimental.pallas import tpu as pltpu
from jax.experimental.pallas import tpu_sc as plsc

DIM = 64
SCALE = 8.0

NC = 2
NS = 16
NW = NC * NS

CHUNK = 256
LANES = 16
NBUF = 2


def _body(x_hbm, table_hbm, out_hbm, idx_v, in_bufs, out_bufs,
          sems_in, sems_out):
    wid = lax.axis_index("s") * NC + lax.axis_index("c")
    n_chunks = x_hbm.shape[1]
    row_base = wid * (n_chunks * CHUNK)

    pltpu.sync_copy(x_hbm.at[wid], idx_v)

    def gather(j, b):
        return pltpu.make_async_copy(
            table_hbm.at[idx_v.at[j]], in_bufs[b], sems_in[b])

    def write(j, b):
        return pltpu.make_async_copy(
            out_bufs[b], out_hbm.at[pl.ds(row_base + j * CHUNK, CHUNK)],
            sems_out[b])

    for b in range(NBUF):
        gather(b, b).start()

    def wave(w, _):
        for b in range(NBUF):
            j = w * NBUF + b
            gather(j, b).wait()

            @pl.when(w > 0)
            def _():
                write(j - NBUF, b).wait()

            def scale_row(i, _):
                for c in range(DIM // LANES):
                    sl = pl.ds(c * LANES, LANES)
                    out_bufs[b][i, sl] = in_bufs[b][i, sl] * SCALE
                return 0

            write(j, b).start()

            @pl.when(j + NBUF < n_chunks)
            def _():
                gather(j + NBUF, b).start()
        return 0

    lax.fori_loop(0, n_chunks // NBUF, wave, 0)

    for b in range(NBUF):
        write(n_chunks - NBUF + b, b).wait()


def kernel(x, table):
    b0, b1 = x.shape
    total = b0 * b1
    n_chunks = total // (NW * CHUNK)
    xf = x.reshape(NW, n_chunks, CHUNK).astype(jnp.int32)

    mesh = plsc.VectorSubcoreMesh(core_axis_name="c", subcore_axis_name="s")
    run = pl.kernel(
        _body,
        out_type=jax.ShapeDtypeStruct((total, DIM), jnp.float32),
        mesh=mesh,
        scratch_types=[
            pltpu.VMEM((n_chunks, CHUNK), jnp.int32),
            [pltpu.VMEM((CHUNK, DIM), jnp.float32) for _ in range(NBUF)],
            [pltpu.VMEM((CHUNK, DIM), jnp.float32) for _ in range(NBUF)],
            [pltpu.SemaphoreType.DMA for _ in range(NBUF)],
            [pltpu.SemaphoreType.DMA for _ in range(NBUF)],
        ],
        compiler_params=pltpu.CompilerParams(use_tc_tiling_on_sc=False),
    )
    out = run(xf, table)
    return out.reshape(b0, b1, DIM)


# speedup vs baseline: 1.3079x; 1.0314x over previous
import jax
import jax.numpy as jnp
from jax import lax
from jax.experimental import pallas as pl
from jax.experimental.pallas import tpu as pltpu
from jax.experimental.pallas import tpu_sc as plsc

DIM = 64
SCALE = 8.0

NC = 2
NS = 16
NW = NC * NS

CHUNK = 128
LANES = 16
NBUF = 4


def _body(x_hbm, table_hbm, out_hbm, idx_v, in_bufs, out_bufs,
          sems_in, sems_out):
    wid = lax.axis_index("s") * NC + lax.axis_index("c")
    n_chunks = x_hbm.shape[1]
    row_base = wid * (n_chunks * CHUNK)

    pltpu.sync_copy(x_hbm.at[wid], idx_v)

    def gather(j, b):
        return pltpu.make_async_copy(
            table_hbm.at[idx_v.at[j]], in_bufs[b], sems_in[b])

    def write(j, b):
        return pltpu.make_async_copy(
            out_bufs[b],
            out_hbm.at[pl.ds(row_base + j * CHUNK, CHUNK), pl.ds(0, DIM)],
            sems_out[b])

    for b in range(NBUF):
        gather(b, b).start()

    def wave(w, _):
        for b in range(NBUF):
            j = w * NBUF + b
            gather(j, b).wait()

            @pl.when(w > 0)
            def _():
                write(j - NBUF, b).wait()

            def scale_row(i, _):
                for c in range(DIM // LANES):
                    sl = pl.ds(c * LANES, LANES)
                    out_bufs[b][i, sl] = in_bufs[b][i, sl] * SCALE
                return 0
            lax.fori_loop(0, CHUNK, scale_row, 0, unroll=8)

            write(j, b).start()

            @pl.when(j + NBUF < n_chunks)
            def _():
                gather(j + NBUF, b).start()
        return 0

    lax.fori_loop(0, n_chunks // NBUF, wave, 0)

    for b in range(NBUF):
        write(n_chunks - NBUF + b, b).wait()


def kernel(x, table):
    b0, b1 = x.shape
    total = b0 * b1
    n_chunks = total // (NW * CHUNK)

    table2 = jnp.pad(table, ((0, 0), (0, DIM))).reshape(-1, DIM)
    xf = (x.astype(jnp.int32) * 2).reshape(NW, n_chunks, CHUNK)

    mesh = plsc.VectorSubcoreMesh(core_axis_name="c", subcore_axis_name="s")
    run = pl.kernel(
        _body,
        out_type=jax.ShapeDtypeStruct((total, 2 * DIM), jnp.float32),
        mesh=mesh,
        scratch_types=[
            pltpu.VMEM((n_chunks, CHUNK), jnp.int32),
            [pltpu.VMEM((CHUNK, DIM), jnp.float32) for _ in range(NBUF)],
            [pltpu.VMEM((CHUNK, DIM), jnp.float32) for _ in range(NBUF)],
            [pltpu.SemaphoreType.DMA for _ in range(NBUF)],
            [pltpu.SemaphoreType.DMA for _ in range(NBUF)],
        ],
        compiler_params=pltpu.CompilerParams(use_tc_tiling_on_sc=False),
    )
    out = run(xf, table2)
    return out[:, :DIM].reshape(b0, b1, DIM)


# speedup vs baseline: 1.8073x; 1.3818x over previous
import jax
import jax.numpy as jnp
from jax import lax
from jax.experimental import pallas as pl
from jax.experimental.pallas import tpu as pltpu
from jax.experimental.pallas import tpu_sc as plsc

DIM = 64
SCALE = 8.0

NC = 2
NS = 16
NW = NC * NS

CHUNK = 128
NBUF = 5
LAG = 2


def _body(x_hbm, table_hbm, out_hbm, idx_v, bufs, sems_in, sems_out):
    wid = lax.axis_index("s") * NC + lax.axis_index("c")
    n_chunks = x_hbm.shape[1]
    row_base = wid * (n_chunks * CHUNK)

    pltpu.sync_copy(x_hbm.at[wid], idx_v)

    def gather(j, b):
        return pltpu.make_async_copy(
            table_hbm.at[idx_v.at[j]], bufs[b], sems_in[b])

    def write(j, b):
        return pltpu.make_async_copy(
            bufs[b],
            out_hbm.at[pl.ds(row_base + j * CHUNK, CHUNK), pl.ds(0, DIM)],
            sems_out[b])

    for b in range(NBUF):
        gather(b, b).start()

    def wave(w, _):
        for b in range(NBUF):
            j = w * NBUF + b
            gather(j, b).wait()
            write(j, b).start()

            jn = j - LAG + NBUF
            bp = (b - LAG) % NBUF

            @pl.when(jnp.logical_and(j >= LAG, jn < n_chunks))
            def _():
                write(j - LAG, bp).wait()
                gather(jn, bp).start()
        return 0

    lax.fori_loop(0, n_chunks // NBUF, wave, 0)

    for k in range(n_chunks - NBUF, n_chunks):
        write(k, k % NBUF).wait()


def kernel(x, table):
    b0, b1 = x.shape
    total = b0 * b1
    n_chunks = total // (NW * CHUNK)

    table2 = (jnp.pad(table, ((0, 0), (0, DIM))) * SCALE).reshape(-1, DIM)
    xf = (x.astype(jnp.int32) * 2).reshape(NW, n_chunks, CHUNK)

    mesh = plsc.VectorSubcoreMesh(core_axis_name="c", subcore_axis_name="s")
    run = pl.kernel(
        _body,
        out_type=jax.ShapeDtypeStruct((total, 2 * DIM), jnp.float32),
        mesh=mesh,
        scratch_types=[
            pltpu.VMEM((n_chunks, CHUNK), jnp.int32),
            [pltpu.VMEM((CHUNK, DIM), jnp.float32) for _ in range(NBUF)],
            [pltpu.SemaphoreType.DMA for _ in range(NBUF)],
            [pltpu.SemaphoreType.DMA for _ in range(NBUF)],
        ],
        compiler_params=pltpu.CompilerParams(use_tc_tiling_on_sc=False),
    )
    out = run(xf, table2)
    return out[:, :DIM].reshape(b0, b1, DIM)
